# Initial kernel scaffold; baseline (speedup 1.0000x reference)
import jax
import jax.numpy as jnp
from jax.experimental import pallas as pl


def kernel(y, edge_index, edge_weight, W0, W1, W2, bias, gamma, beta):
    raise NotImplementedError("write your pallas kernel here")


# speedup vs baseline: 2.8433x; 2.8433x over previous
import jax
import jax.numpy as jnp
from jax import lax
from jax.experimental import pallas as pl
from jax.experimental.pallas import tpu as pltpu
from jax.experimental.pallas import tpu_sc as plsc

N = 10000
E = 320000
D = 128

NC = 2
NS = 16
L = 16
NW = NC * NS
EPW = E // NW
CB = 128
GCH = 8
SGB = GCH * CB
NSG = 10
EPWP = NSG * SGB
NP = 10112
RPT = NP // NS

_mesh = plsc.VectorSubcoreMesh(core_axis_name="c", subcore_axis_name="s")


def _sc_hop_body(tab_hbm, src_hbm, dst_hbm, ew_hbm, zeros_hbm, out_hbm,
                 acc, src_v, dst_v, ew_v, buf):
    c = lax.axis_index("c")
    t = lax.axis_index("s")
    w = c * NS + t
    r0 = t * RPT

    pltpu.sync_copy(zeros_hbm, acc.at[pl.ds(r0, RPT)])
    plsc.subcore_barrier()

    def group(g, carry):
        pltpu.sync_copy(src_hbm.at[w].at[g], src_v)
        pltpu.sync_copy(dst_hbm.at[w].at[g], dst_v)
        pltpu.sync_copy(ew_hbm.at[w].at[g].at[0], ew_v)

        def chunk(j, c1):
            pltpu.sync_copy(tab_hbm.at[src_v.at[j]], buf)

            def edge(i, c2):
                wv = plsc.load_gather(
                    ew_v, [jnp.full((L,), j * CB + i, jnp.int32)])
                for q in range(D // L):
                    sl = (i, pl.ds(q * L, L))
                    buf[sl] = buf[sl] * wv
                return c2

            lax.fori_loop(0, CB, edge, 0, unroll=8)
            pltpu.sync_copy(buf, acc.at[dst_v.at[j]], add=True)
            return c1

        lax.fori_loop(0, GCH, chunk, 0)
        return carry

    lax.fori_loop(0, NSG, group, 0)
    plsc.subcore_barrier()

    pltpu.sync_copy(acc.at[pl.ds(r0, RPT)], out_hbm.at[c].at[pl.ds(r0, RPT)])


_sc_hop = pl.kernel(
    _sc_hop_body,
    out_type=jax.ShapeDtypeStruct((NC, NP, D), jnp.float32),
    mesh=_mesh,
    compiler_params=pltpu.CompilerParams(needs_layout_passes=False),
    scratch_types=[
        pltpu.VMEM_SHARED((NP, D), jnp.float32),
        pltpu.VMEM((GCH, CB), jnp.int32),
        pltpu.VMEM((GCH, CB), jnp.int32),
        pltpu.VMEM((SGB,), jnp.float32),
        pltpu.VMEM((CB, D), jnp.float32),
    ],
)


def _tc_sum_body(p_ref, out_ref):
    out_ref[...] = p_ref[0] + p_ref[1]


_tc_sum = pl.pallas_call(
    _tc_sum_body,
    out_shape=jax.ShapeDtypeStruct((NP, D), jnp.float32),
)


def _tc_body(y_ref, x1_ref, p2_ref, w0_ref, w1_ref, w2_ref, b_ref, g_ref,
             be_ref, out_ref):
    y = y_ref[...]
    x2 = p2_ref[0, :N, :] + p2_ref[1, :N, :]
    h = jnp.dot(y, w0_ref[...], preferred_element_type=jnp.float32)
    h += jnp.dot(x1_ref[:N, :], w1_ref[...], preferred_element_type=jnp.float32)
    h += jnp.dot(x2, w2_ref[...], preferred_element_type=jnp.float32)
    h += b_ref[...]
    mean = jnp.mean(h, axis=0, keepdims=True)
    var = jnp.mean(jnp.square(h - mean), axis=0, keepdims=True)
    hn = (h - mean) * lax.rsqrt(var + 1e-5)
    hb = g_ref[...] * hn + be_ref[...]
    out_ref[...] = y + jnp.where(hb >= 0, hb, 0.01 * hb)


_tc_call = pl.pallas_call(
    _tc_body,
    out_shape=jax.ShapeDtypeStruct((N, D), jnp.float32),
)


@jax.jit
def kernel(y, edge_index, edge_weight, W0, W1, W2, bias, gamma, beta):
    y_pad = jnp.pad(y, ((0, NP - N), (0, 0)))
    pad = ((0, 0), (0, EPWP - EPW))
    src = jnp.pad(edge_index[0].reshape(NW, EPW), pad, constant_values=N)
    dst = jnp.pad(edge_index[1].reshape(NW, EPW), pad, constant_values=N)
    ew = jnp.pad(edge_weight.reshape(NW, EPW), pad)
    src = src.reshape(NW, NSG, GCH, CB)
    dst = dst.reshape(NW, NSG, GCH, CB)
    ew = ew.reshape(NW, NSG, 1, SGB)
    zeros = jnp.zeros((RPT, D), jnp.float32)
    p1 = _sc_hop(y_pad, src, dst, ew, zeros)
    x1p = _tc_sum(p1)
    p2 = _sc_hop(x1p, src, dst, ew, zeros)
    return _tc_call(y, x1p, p2, W0, W1, W2,
                    bias.reshape(1, D), gamma.reshape(1, D),
                    beta.reshape(1, D))
